# speedup vs baseline: 1.3496x; 1.3496x over previous
import jax
import jax.numpy as jnp
from jax.experimental import pallas as pl

B, S, D = 1, 2048, 1024
E, TOPK, NZ, FF, ZH = 16, 2, 8, 512, 256
T = B * S


def _routing_body(x_ref, gu_ref, gw_ref, zw1_ref, zb1_ref, zw2_ref, zb2_ref,
                  u_ref, comb_ref):
    x = x_ref[...]
    rl = jax.lax.dot_general(x, gw_ref[...], (((1,), (1,)), ((), ())),
                             preferred_element_type=jnp.float32)
    h1 = jax.lax.dot_general(x, zw1_ref[...], (((1,), (1,)), ((), ())),
                             preferred_element_type=jnp.float32)
    h1 = h1 + zb1_ref[...]
    h1 = h1 * jax.nn.sigmoid(h1)
    zl = jax.lax.dot_general(h1, zw2_ref[...], (((1,), (1,)), ((), ())),
                             preferred_element_type=jnp.float32)
    zl = zl + zb2_ref[...]
    g = -jnp.log(-jnp.log(gu_ref[...]))
    s = zl + g
    iota_nz = jax.lax.broadcasted_iota(jnp.int32, (T, NZ), 1)
    smax = jnp.max(s, axis=1, keepdims=True)
    idx = jnp.min(jnp.where(s >= smax, iota_nz, NZ), axis=1, keepdims=True)
    oh = jnp.where(iota_nz == idx, 1.0, 0.0).astype(jnp.float32)
    bias = jnp.dot(oh, u_ref[...], preferred_element_type=jnp.float32)
    logits = rl + bias
    lmax = jnp.max(logits, axis=1, keepdims=True)
    ex = jnp.exp(logits - lmax)
    rw = ex / jnp.sum(ex, axis=1, keepdims=True)
    iota_e = jax.lax.broadcasted_iota(jnp.int32, (T, E), 1)
    w0 = jnp.max(rw, axis=1, keepdims=True)
    e0 = jnp.min(jnp.where(rw >= w0, iota_e, E), axis=1, keepdims=True)
    rw2 = jnp.where(iota_e == e0, -1.0, rw)
    w1 = jnp.max(rw2, axis=1, keepdims=True)
    e1 = jnp.min(jnp.where(rw2 >= w1, iota_e, E), axis=1, keepdims=True)
    comb = jnp.where(iota_e == e0, w0, 0.0) + jnp.where(iota_e == e1, w1, 0.0)
    comb_ref[...] = comb.astype(jnp.float32)


def _routing(x, gumbel_u, gate_weight, zW1, zb1, zW2, zb2, U):
    return pl.pallas_call(
        _routing_body,
        out_shape=jax.ShapeDtypeStruct((T, E), jnp.float32),
    )(x, gumbel_u, gate_weight, zW1, zb1.reshape(1, ZH), zW2,
      zb2.reshape(1, NZ), U)


def _experts_body(xb_ref, comb_ref, wg_ref, wu_ref, wd_ref, out_ref):
    e = pl.program_id(0)

    @pl.when(e == 0)
    def _():
        out_ref[...] = jnp.zeros_like(out_ref)

    xb = xb_ref[...]
    gp = jax.lax.dot_general(xb, wg_ref[0], (((1,), (1,)), ((), ())),
                             preferred_element_type=jnp.float32)
    up = jax.lax.dot_general(xb, wu_ref[0], (((1,), (1,)), ((), ())),
                             preferred_element_type=jnp.float32)
    hm = (gp * jax.nn.sigmoid(gp) * up).astype(jnp.bfloat16)
    y = jax.lax.dot_general(hm, wd_ref[0], (((1,), (1,)), ((), ())),
                            preferred_element_type=jnp.float32)
    iota_e = jax.lax.broadcasted_iota(jnp.int32, (T, E), 1)
    ce = jnp.sum(jnp.where(iota_e == e, comb_ref[...], 0.0), axis=1,
                 keepdims=True)
    out_ref[...] += ce * y


def _experts(xb, comb, Wg, Wu, Wd):
    return pl.pallas_call(
        _experts_body,
        grid=(E,),
        in_specs=[
            pl.BlockSpec((T, D), lambda e: (0, 0)),
            pl.BlockSpec((T, E), lambda e: (0, 0)),
            pl.BlockSpec((1, FF, D), lambda e: (e, 0, 0)),
            pl.BlockSpec((1, FF, D), lambda e: (e, 0, 0)),
            pl.BlockSpec((1, D, FF), lambda e: (e, 0, 0)),
        ],
        out_specs=pl.BlockSpec((T, D), lambda e: (0, 0)),
        out_shape=jax.ShapeDtypeStruct((T, D), jnp.float32),
    )(xb, comb, Wg, Wu, Wd)


def kernel(hidden_states, gumbel_u, gate_weight, zW1, zb1, zW2, zb2, U, Wg,
           Wu, Wd):
    x = hidden_states.reshape(T, D)
    comb = _routing(x, gumbel_u, gate_weight, zW1, zb1, zW2, zb2, U)
    xb = x.astype(jnp.bfloat16)
    out = _experts(xb, comb, Wg.astype(jnp.bfloat16), Wu.astype(jnp.bfloat16),
                   Wd.astype(jnp.bfloat16))
    return out.reshape(B, S, D)
